# baseline (speedup 1.0000x reference)
import functools

import jax
import jax.numpy as jnp
from jax import lax
from jax.experimental import pallas as pl
from jax.experimental.pallas import tpu as pltpu
from jax.experimental.pallas import tpu_sc as plsc

N_NODES = 10000
N_EDGES = 320000
EMB = 128
N_LAYERS = 5
N_GRAPHS = 64
OUT = 128
ID_VOCAB = 50

NC = 2
NS = 16
NW = NC * NS
E_PER_TILE = N_EDGES // NW
CW0, NCH0 = 96, 105
CW1, NCH1 = 128, 79
AGG_ROWS = N_NODES + 8
PACK_SHIFT = 14
PACK_MASK = (1 << PACK_SHIFT) - 1
ROWS_PER_TILE = 624
ZTAIL = AGG_ROWS - NS * ROWS_PER_TILE
OTAIL = N_NODES - NS * ROWS_PER_TILE

F32 = jnp.float32
HI = lax.Precision.DEFAULT


def _sc_mesh():
    return plsc.VectorSubcoreMesh(
        core_axis_name="c", subcore_axis_name="s", num_cores=NC, num_subcores=NS)


def _zero_vmem_block(buf, nrows):
    zero = jnp.zeros((16,), F32)

    def row(i, _):
        for k in range(EMB // 16):
            buf[i, pl.ds(k * 16, 16)] = zero
        return 0

    lax.fori_loop(0, nrows, row, 0)


def _zero_agg_stripe(buf, cw, agg, s):
    stripe = s * ROWS_PER_TILE
    n_full = ROWS_PER_TILE // cw
    rem = ROWS_PER_TILE - n_full * cw
    for g in range(n_full):
        pltpu.sync_copy(buf, agg.at[pl.ds(stripe + g * cw, cw)])
    if rem:
        pltpu.sync_copy(buf.at[pl.ds(0, rem)],
                        agg.at[pl.ds(stripe + n_full * cw, rem)])

    @pl.when(s == NS - 1)
    def _():
        pltpu.sync_copy(buf.at[pl.ds(0, ZTAIL)],
                        agg.at[pl.ds(NS * ROWS_PER_TILE, ZTAIL)])


def _copy_out_stripe(agg, out_hbm, c, s):
    stripe = s * ROWS_PER_TILE
    pltpu.sync_copy(agg.at[pl.ds(stripe, ROWS_PER_TILE)],
                    out_hbm.at[c, pl.ds(stripe, ROWS_PER_TILE)])

    @pl.when(s == NS - 1)
    def _():
        pltpu.sync_copy(agg.at[pl.ds(NS * ROWS_PER_TILE, OTAIL)],
                        out_hbm.at[c, pl.ds(NS * ROWS_PER_TILE, OTAIL)])


def _unpack_chunk(packed_i, g, cw, hi_v, lo_v):
    for k in range(cw // 16):
        sl = pl.ds(k * 16, 16)
        v = packed_i[pl.ds(g * cw + k * 16, 16)]
        hi_v[sl] = lax.shift_right_logical(v, PACK_SHIFT)
        lo_v[sl] = lax.bitwise_and(v, PACK_MASK)


def _relu_add_rows(xrows, trows, nrows):
    def row(i, _):
        for kk in range(EMB // 16):
            sl = pl.ds(kk * 16, 16)
            v = xrows[i, sl] + trows[i, sl]
            xrows[i, sl] = jnp.maximum(v, 0.0)
        return 0

    lax.fori_loop(0, nrows, row, 0)


def _edge_layer0_call():
    @functools.partial(
        pl.kernel,
        out_type=jax.ShapeDtypeStruct((NC, N_NODES, EMB), F32),
        mesh=_sc_mesh(),
        scratch_types=[
            pltpu.VMEM((NCH0 * CW0,), jnp.int32),
            pltpu.VMEM((NCH0 * CW0,), jnp.int32),
            pltpu.VMEM((CW0,), jnp.int32),
            pltpu.VMEM((CW0,), jnp.int32),
            pltpu.VMEM((CW0, EMB), F32),
            pltpu.VMEM((CW0, EMB), F32),
            pltpu.VMEM_SHARED((AGG_ROWS, EMB), F32),
            pltpu.SemaphoreType.DMA,
            pltpu.SemaphoreType.DMA,
        ],
    )
    def k(x_hbm, t_hbm, src_hbm, pcd_hbm, out_hbm,
          src_i, pcd_i, ci, di, xr, tr, agg, semx, semt):
        c = lax.axis_index("c")
        s = lax.axis_index("s")
        w = c * NS + s

        pltpu.sync_copy(src_hbm.at[w], src_i)
        pltpu.sync_copy(pcd_hbm.at[w], pcd_i)
        _zero_vmem_block(xr, CW0)
        _zero_agg_stripe(xr, CW0, agg, s)
        plsc.subcore_barrier()

        def body(g, _):
            _unpack_chunk(pcd_i, g, CW0, ci, di)
            pltpu.async_copy(x_hbm.at[src_i.at[pl.ds(g * CW0, CW0)]],
                             xr, semx)
            pltpu.async_copy(t_hbm.at[ci], tr, semt)
            pltpu.make_async_copy(
                x_hbm.at[src_i.at[pl.ds(g * CW0, CW0)]], xr, semx).wait()
            pltpu.make_async_copy(t_hbm.at[ci], tr, semt).wait()
            _relu_add_rows(xr, tr, CW0)
            pltpu.sync_copy(xr, agg.at[di], add=True)
            return 0

        lax.fori_loop(0, NCH0, body, 0)
        plsc.subcore_barrier()
        _copy_out_stripe(agg, out_hbm, c, s)

    return k


def _edge_gather_scatter_call():
    @functools.partial(
        pl.kernel,
        out_type=jax.ShapeDtypeStruct((NC, N_NODES, EMB), F32),
        mesh=_sc_mesh(),
        scratch_types=[
            pltpu.VMEM((NCH1 * CW1,), jnp.int32),
            pltpu.VMEM((CW1,), jnp.int32),
            pltpu.VMEM((CW1,), jnp.int32),
            pltpu.VMEM((CW1, EMB), F32),
            pltpu.VMEM_SHARED((AGG_ROWS, EMB), F32),
            pltpu.SemaphoreType.DMA,
        ],
    )
    def k(y_hbm, pyd_hbm, out_hbm, pyd_i, yidx_v, dst_v, rows, agg, sem):
        c = lax.axis_index("c")
        s = lax.axis_index("s")
        w = c * NS + s

        pltpu.sync_copy(pyd_hbm.at[w], pyd_i)
        _zero_vmem_block(rows, CW1)
        _zero_agg_stripe(rows, CW1, agg, s)
        plsc.subcore_barrier()

        def body(g, _):
            _unpack_chunk(pyd_i, g, CW1, yidx_v, dst_v)
            pltpu.async_copy(y_hbm.at[yidx_v], rows, sem).wait()
            pltpu.sync_copy(rows, agg.at[dst_v], add=True)
            return 0

        lax.fori_loop(0, NCH1, body, 0)
        plsc.subcore_barrier()
        _copy_out_stripe(agg, out_hbm, c, s)

    return k


ROW_BLK = 1000
N_BLKS = N_NODES // ROW_BLK


def _h_kernel(xn_pad, d_pad, base):
    def body(xn_ref, d_ref, b_ref, o_ref):
        o_ref[...] = (
            jnp.dot(xn_ref[...], d_ref[...], preferred_element_type=F32,
                    precision=HI)
            + b_ref[...])

    return pl.pallas_call(
        body,
        grid=(N_BLKS,),
        in_specs=[
            pl.BlockSpec((ROW_BLK, 16), lambda i: (i, 0)),
            pl.BlockSpec((16, EMB), lambda i: (0, 0)),
            pl.BlockSpec((1, EMB), lambda i: (0, 0)),
        ],
        out_specs=pl.BlockSpec((ROW_BLK, EMB), lambda i: (i, 0)),
        out_shape=jax.ShapeDtypeStruct((N_NODES, EMB), F32),
    )(xn_pad, d_pad, base)


def _mlp_kernel(x, agg, W1l, b1l, W2l, b2l, t_next, relu_out, emit_y):
    def body(x_ref, a_ref, w1_ref, b1_ref, w2_ref, b2_ref, t_ref,
             xo_ref, yo_ref=None):
        u = x_ref[...] + a_ref[0] + a_ref[1]
        h1 = jnp.maximum(
            jnp.dot(u, w1_ref[...], preferred_element_type=F32, precision=HI)
            + b1_ref[...], 0.0)
        o = (jnp.dot(h1, w2_ref[...], preferred_element_type=F32, precision=HI)
             + b2_ref[...])
        if relu_out:
            o = jnp.maximum(o, 0.0)
        xo_ref[...] = o
        if emit_y:
            for cc in range(8):
                yo_ref[cc] = jnp.maximum(o + t_ref[cc], 0.0)

    out_shapes = [jax.ShapeDtypeStruct((N_NODES, EMB), F32)]
    out_specs = [pl.BlockSpec((ROW_BLK, EMB), lambda i: (i, 0))]
    if emit_y:
        out_shapes.append(jax.ShapeDtypeStruct((8, N_NODES, EMB), F32))
        out_specs.append(pl.BlockSpec((8, ROW_BLK, EMB), lambda i: (0, i, 0)))

    if emit_y:
        wrapped = body
    else:
        def wrapped(x_ref, a_ref, w1_ref, b1_ref, w2_ref, b2_ref, t_ref,
                    xo_ref):
            body(x_ref, a_ref, w1_ref, b1_ref, w2_ref, b2_ref, t_ref, xo_ref)

    res = pl.pallas_call(
        wrapped,
        grid=(N_BLKS,),
        in_specs=[
            pl.BlockSpec((ROW_BLK, EMB), lambda i: (i, 0)),
            pl.BlockSpec((NC, ROW_BLK, EMB), lambda i: (0, i, 0)),
            pl.BlockSpec((EMB, 2 * EMB), lambda i: (0, 0)),
            pl.BlockSpec((1, 2 * EMB), lambda i: (0, 0)),
            pl.BlockSpec((2 * EMB, EMB), lambda i: (0, 0)),
            pl.BlockSpec((1, EMB), lambda i: (0, 0)),
            pl.BlockSpec((8, EMB), lambda i: (0, 0)),
        ],
        out_specs=out_specs,
        out_shape=out_shapes,
    )(x, agg, W1l, b1l.reshape(1, -1), W2l, b2l.reshape(1, -1), t_next)
    if emit_y:
        return res[0], res[1]
    return res[0], None


def _pool_kernel(x, batch_r, proj_W, proj_b):
    def body(x_ref, b_ref, pw_ref, pb_ref, o_ref, sums, counts):
        i = pl.program_id(0)

        @pl.when(i == 0)
        def _():
            sums[...] = jnp.zeros_like(sums)
            counts[...] = jnp.zeros_like(counts)

        gid = lax.broadcasted_iota(jnp.int32, (N_GRAPHS, ROW_BLK), 0)
        oh = (jnp.broadcast_to(b_ref[0], (N_GRAPHS, ROW_BLK)) == gid
              ).astype(F32)
        sums[...] += jnp.dot(oh, x_ref[...], preferred_element_type=F32,
                             precision=HI)
        counts[...] += jnp.broadcast_to(
            jnp.sum(oh, axis=1, keepdims=True), (N_GRAPHS, EMB))

        @pl.when(i == N_BLKS - 1)
        def _():
            pooled = sums[...] / jnp.maximum(counts[...], 1.0)
            o_ref[...] = (
                jnp.dot(pooled, pw_ref[...], preferred_element_type=F32,
                        precision=HI)
                + pb_ref[...])

    return pl.pallas_call(
        body,
        grid=(N_BLKS,),
        in_specs=[
            pl.BlockSpec((ROW_BLK, EMB), lambda i: (i, 0)),
            pl.BlockSpec((1, 1, ROW_BLK), lambda i: (i, 0, 0)),
            pl.BlockSpec((EMB, OUT), lambda i: (0, 0)),
            pl.BlockSpec((1, OUT), lambda i: (0, 0)),
        ],
        out_specs=pl.BlockSpec((N_GRAPHS, OUT), lambda i: (0, 0)),
        out_shape=jax.ShapeDtypeStruct((N_GRAPHS, OUT), F32),
        scratch_shapes=[
            pltpu.VMEM((N_GRAPHS, EMB), F32),
            pltpu.VMEM((N_GRAPHS, EMB), F32),
        ],
    )(x, batch_r, proj_W, proj_b.reshape(1, -1))


def kernel(x_nodes, edge_index, degrees, identifiers, edge_features, batch,
           atom_emb, id_emb, bond_emb, W1, b1, W2, b2, proj_W, proj_b):
    del degrees

    base = atom_emb[:, 0, :].sum(axis=0).reshape(1, EMB)
    diff = atom_emb[:, 1, :] - atom_emb[:, 0, :]
    d_pad = jnp.zeros((16, EMB), F32).at[:9].set(diff)
    xn_pad = jnp.zeros((N_NODES, 16), F32).at[:, :9].set(
        x_nodes.astype(F32))

    bits = jnp.array([[c & 1, (c >> 1) & 1, (c >> 2) & 1] for c in range(8)],
                     dtype=jnp.int32)
    t_combo = (bond_emb[:, 0, bits[:, 0], :]
               + bond_emb[:, 1, bits[:, 1], :]
               + bond_emb[:, 2, bits[:, 2], :])

    combo = (edge_features[:, 0] + 2 * edge_features[:, 1]
             + 4 * edge_features[:, 2]).astype(jnp.int32)
    src = edge_index[0]
    dst = edge_index[1]

    t0 = (id_emb[:, None, :] + t_combo[0][None, :, :]).reshape(
        ID_VOCAB * 8, EMB)
    cidx0 = identifiers[:, 0] * 8 + combo
    yidx = combo * N_NODES + src

    def pad_tiles(a, nch, cw, pad_val):
        a2 = a.reshape(NW, E_PER_TILE)
        pad = jnp.full((NW, nch * cw - E_PER_TILE), pad_val, jnp.int32)
        return jnp.concatenate([a2, pad], axis=1)

    src_r = pad_tiles(src, NCH0, CW0, 0)
    pcd_r = pad_tiles((cidx0 << PACK_SHIFT) | dst, NCH0, CW0, N_NODES)
    pyd_r = pad_tiles((yidx << PACK_SHIFT) | dst, NCH1, CW1, N_NODES)

    batch_r = batch.reshape(N_BLKS, 1, ROW_BLK)

    sc_l0 = _edge_layer0_call()
    sc_gs = _edge_gather_scatter_call()

    x = _h_kernel(xn_pad, d_pad, base)

    agg = sc_l0(x, t0, src_r, pcd_r)
    x, y = _mlp_kernel(x, agg, W1[0], b1[0], W2[0], b2[0], t_combo[1],
                       relu_out=True, emit_y=True)

    for l in range(1, N_LAYERS):
        agg = sc_gs(y.reshape(8 * N_NODES, EMB), pyd_r)
        last = l == N_LAYERS - 1
        t_next = t_combo[l + 1] if not last else t_combo[0]
        x, y = _mlp_kernel(x, agg, W1[l], b1[l], W2[l], b2[l], t_next,
                           relu_out=not last, emit_y=not last)

    return _pool_kernel(x, batch_r, proj_W, proj_b)


# speedup vs baseline: 1.4227x; 1.4227x over previous
import functools

import jax
import jax.numpy as jnp
from jax import lax
from jax.experimental import pallas as pl
from jax.experimental.pallas import tpu as pltpu
from jax.experimental.pallas import tpu_sc as plsc

N_NODES = 10000
N_EDGES = 320000
EMB = 128
N_LAYERS = 5
N_GRAPHS = 64
OUT = 128
ID_VOCAB = 50

NC = 2
NS = 16
NW = NC * NS
E_PER_TILE = N_EDGES // NW
CW0, NCH0 = 96, 105
CW1 = 128
CC1 = 2 * CW1
NCC1 = N_EDGES // CC1
NCC1_BASE = NCC1 // NW
AGG_ROWS = N_NODES + 8
PACK_SHIFT = 14
PACK_MASK = (1 << PACK_SHIFT) - 1
ROWS_PER_TILE = 624
ZTAIL = AGG_ROWS - NS * ROWS_PER_TILE
OTAIL = N_NODES - NS * ROWS_PER_TILE

F32 = jnp.float32
HI = lax.Precision.DEFAULT


def _sc_mesh():
    return plsc.VectorSubcoreMesh(
        core_axis_name="c", subcore_axis_name="s", num_cores=NC, num_subcores=NS)


def _zero_vmem_block(buf, nrows):
    zero = jnp.zeros((16,), F32)

    def row(i, _):
        for k in range(EMB // 16):
            buf[i, pl.ds(k * 16, 16)] = zero
        return 0

    lax.fori_loop(0, nrows, row, 0)


def _zero_agg_stripe(buf, cw, agg, s):
    stripe = s * ROWS_PER_TILE
    n_full = ROWS_PER_TILE // cw
    rem = ROWS_PER_TILE - n_full * cw
    for g in range(n_full):
        pltpu.sync_copy(buf, agg.at[pl.ds(stripe + g * cw, cw)])
    if rem:
        pltpu.sync_copy(buf.at[pl.ds(0, rem)],
                        agg.at[pl.ds(stripe + n_full * cw, rem)])

    @pl.when(s == NS - 1)
    def _():
        pltpu.sync_copy(buf.at[pl.ds(0, ZTAIL)],
                        agg.at[pl.ds(NS * ROWS_PER_TILE, ZTAIL)])


def _copy_out_stripe(agg, out_hbm, c, s):
    stripe = s * ROWS_PER_TILE
    pltpu.sync_copy(agg.at[pl.ds(stripe, ROWS_PER_TILE)],
                    out_hbm.at[c, pl.ds(stripe, ROWS_PER_TILE)])

    @pl.when(s == NS - 1)
    def _():
        pltpu.sync_copy(agg.at[pl.ds(NS * ROWS_PER_TILE, OTAIL)],
                        out_hbm.at[c, pl.ds(NS * ROWS_PER_TILE, OTAIL)])


def _unpack_chunk(packed_i, g, cw, hi_v, lo_v):
    for k in range(cw // 16):
        sl = pl.ds(k * 16, 16)
        v = packed_i[pl.ds(g * cw + k * 16, 16)]
        hi_v[sl] = lax.shift_right_logical(v, PACK_SHIFT)
        lo_v[sl] = lax.bitwise_and(v, PACK_MASK)


def _relu_add_rows(xrows, trows, nrows):
    def row(i, _):
        for kk in range(EMB // 16):
            sl = pl.ds(kk * 16, 16)
            v = xrows[i, sl] + trows[i, sl]
            xrows[i, sl] = jnp.maximum(v, 0.0)
        return 0

    lax.fori_loop(0, nrows, row, 0)


def _edge_layer0_call():
    @functools.partial(
        pl.kernel,
        out_type=jax.ShapeDtypeStruct((NC, N_NODES, EMB), F32),
        mesh=_sc_mesh(),
        scratch_types=[
            pltpu.VMEM((NCH0 * CW0,), jnp.int32),
            pltpu.VMEM((NCH0 * CW0,), jnp.int32),
            pltpu.VMEM((CW0,), jnp.int32),
            pltpu.VMEM((CW0,), jnp.int32),
            pltpu.VMEM((CW0, EMB), F32),
            pltpu.VMEM((CW0, EMB), F32),
            pltpu.VMEM_SHARED((AGG_ROWS, EMB), F32),
            pltpu.SemaphoreType.DMA,
            pltpu.SemaphoreType.DMA,
        ],
    )
    def k(x_hbm, t_hbm, src_hbm, pcd_hbm, out_hbm,
          src_i, pcd_i, ci, di, xr, tr, agg, semx, semt):
        c = lax.axis_index("c")
        s = lax.axis_index("s")
        w = c * NS + s

        pltpu.sync_copy(src_hbm.at[w], src_i)
        pltpu.sync_copy(pcd_hbm.at[w], pcd_i)
        _zero_vmem_block(xr, CW0)
        _zero_agg_stripe(xr, CW0, agg, s)
        plsc.subcore_barrier()

        def body(g, _):
            _unpack_chunk(pcd_i, g, CW0, ci, di)
            pltpu.async_copy(x_hbm.at[src_i.at[pl.ds(g * CW0, CW0)]],
                             xr, semx)
            pltpu.async_copy(t_hbm.at[ci], tr, semt)
            pltpu.make_async_copy(
                x_hbm.at[src_i.at[pl.ds(g * CW0, CW0)]], xr, semx).wait()
            pltpu.make_async_copy(t_hbm.at[ci], tr, semt).wait()
            _relu_add_rows(xr, tr, CW0)
            pltpu.sync_copy(xr, agg.at[di], add=True)
            return 0

        lax.fori_loop(0, NCH0, body, 0)
        plsc.subcore_barrier()
        _copy_out_stripe(agg, out_hbm, c, s)

    return k


def _edge_gather_scatter_call():
    @functools.partial(
        pl.kernel,
        out_type=jax.ShapeDtypeStruct((NC, N_NODES, EMB), F32),
        mesh=_sc_mesh(),
        scratch_types=[
            pltpu.VMEM((CC1,), jnp.int32),
            pltpu.VMEM((CW1,), jnp.int32),
            pltpu.VMEM((CW1,), jnp.int32),
            pltpu.VMEM((CW1,), jnp.int32),
            pltpu.VMEM((CW1,), jnp.int32),
            pltpu.VMEM((CW1, EMB), F32),
            pltpu.VMEM((CW1, EMB), F32),
            pltpu.VMEM_SHARED((AGG_ROWS, EMB), F32),
            pltpu.SemaphoreType.DMA,
            pltpu.SemaphoreType.DMA,
        ],
    )
    def k(y_hbm, pyd_hbm, out_hbm, pbuf, yiA, yiB, diA, diB, rA, rB, agg,
          semg, semsc):
        c = lax.axis_index("c")
        s = lax.axis_index("s")
        w = c * NS + s

        _zero_vmem_block(rA, CW1)
        _zero_agg_stripe(rA, CW1, agg, s)
        plsc.subcore_barrier()

        n_my = NCC1_BASE + jnp.where(w < NCC1 - NCC1_BASE * NW, 1, 0)

        def body(g, _):
            @pl.when(g > 0)
            def _():
                pltpu.make_async_copy(rA, agg.at[diA], semsc).wait()
                pltpu.make_async_copy(rB, agg.at[diB], semsc).wait()

            base = (w + g * NW) * CC1
            pltpu.sync_copy(pyd_hbm.at[pl.ds(base, CC1)], pbuf)
            _unpack_chunk(pbuf, 0, CW1, yiA, diA)
            _unpack_chunk(pbuf, 1, CW1, yiB, diB)
            pltpu.async_copy(y_hbm.at[yiA], rA, semg)
            pltpu.async_copy(y_hbm.at[yiB], rB, semg)
            pltpu.make_async_copy(y_hbm.at[yiA], rA, semg).wait()
            pltpu.make_async_copy(y_hbm.at[yiB], rB, semg).wait()
            pltpu.async_copy(rA, agg.at[diA], semsc, add=True)
            pltpu.async_copy(rB, agg.at[diB], semsc, add=True)
            return 0

        lax.fori_loop(0, n_my, body, 0)
        pltpu.make_async_copy(rA, agg.at[diA], semsc).wait()
        pltpu.make_async_copy(rB, agg.at[diB], semsc).wait()
        plsc.subcore_barrier()
        _copy_out_stripe(agg, out_hbm, c, s)

    return k


ROW_BLK = 1000
N_BLKS = N_NODES // ROW_BLK


def _h_kernel(xn_pad, d_pad, base):
    def body(xn_ref, d_ref, b_ref, o_ref):
        o_ref[...] = (
            jnp.dot(xn_ref[...], d_ref[...], preferred_element_type=F32,
                    precision=HI)
            + b_ref[...])

    return pl.pallas_call(
        body,
        grid=(N_BLKS,),
        in_specs=[
            pl.BlockSpec((ROW_BLK, 16), lambda i: (i, 0)),
            pl.BlockSpec((16, EMB), lambda i: (0, 0)),
            pl.BlockSpec((1, EMB), lambda i: (0, 0)),
        ],
        out_specs=pl.BlockSpec((ROW_BLK, EMB), lambda i: (i, 0)),
        out_shape=jax.ShapeDtypeStruct((N_NODES, EMB), F32),
    )(xn_pad, d_pad, base)


def _mlp_kernel(x, agg, W1l, b1l, W2l, b2l, t_next, relu_out, emit_y):
    def body(x_ref, a_ref, w1_ref, b1_ref, w2_ref, b2_ref, t_ref,
             xo_ref, yo_ref=None):
        u = x_ref[...] + a_ref[0] + a_ref[1]
        h1 = jnp.maximum(
            jnp.dot(u, w1_ref[...], preferred_element_type=F32, precision=HI)
            + b1_ref[...], 0.0)
        o = (jnp.dot(h1, w2_ref[...], preferred_element_type=F32, precision=HI)
             + b2_ref[...])
        if relu_out:
            o = jnp.maximum(o, 0.0)
        xo_ref[...] = o
        if emit_y:
            for cc in range(8):
                yo_ref[cc] = jnp.maximum(o + t_ref[cc], 0.0)

    out_shapes = [jax.ShapeDtypeStruct((N_NODES, EMB), F32)]
    out_specs = [pl.BlockSpec((ROW_BLK, EMB), lambda i: (i, 0))]
    if emit_y:
        out_shapes.append(jax.ShapeDtypeStruct((8, N_NODES, EMB), F32))
        out_specs.append(pl.BlockSpec((8, ROW_BLK, EMB), lambda i: (0, i, 0)))

    if emit_y:
        wrapped = body
    else:
        def wrapped(x_ref, a_ref, w1_ref, b1_ref, w2_ref, b2_ref, t_ref,
                    xo_ref):
            body(x_ref, a_ref, w1_ref, b1_ref, w2_ref, b2_ref, t_ref, xo_ref)

    res = pl.pallas_call(
        wrapped,
        grid=(N_BLKS,),
        in_specs=[
            pl.BlockSpec((ROW_BLK, EMB), lambda i: (i, 0)),
            pl.BlockSpec((NC, ROW_BLK, EMB), lambda i: (0, i, 0)),
            pl.BlockSpec((EMB, 2 * EMB), lambda i: (0, 0)),
            pl.BlockSpec((1, 2 * EMB), lambda i: (0, 0)),
            pl.BlockSpec((2 * EMB, EMB), lambda i: (0, 0)),
            pl.BlockSpec((1, EMB), lambda i: (0, 0)),
            pl.BlockSpec((8, EMB), lambda i: (0, 0)),
        ],
        out_specs=out_specs,
        out_shape=out_shapes,
    )(x, agg, W1l, b1l.reshape(1, -1), W2l, b2l.reshape(1, -1), t_next)
    if emit_y:
        return res[0], res[1]
    return res[0], None


def _pool_kernel(x, batch_r, proj_W, proj_b):
    def body(x_ref, b_ref, pw_ref, pb_ref, o_ref, sums, counts):
        i = pl.program_id(0)

        @pl.when(i == 0)
        def _():
            sums[...] = jnp.zeros_like(sums)
            counts[...] = jnp.zeros_like(counts)

        gid = lax.broadcasted_iota(jnp.int32, (N_GRAPHS, ROW_BLK), 0)
        oh = (jnp.broadcast_to(b_ref[0], (N_GRAPHS, ROW_BLK)) == gid
              ).astype(F32)
        sums[...] += jnp.dot(oh, x_ref[...], preferred_element_type=F32,
                             precision=HI)
        counts[...] += jnp.broadcast_to(
            jnp.sum(oh, axis=1, keepdims=True), (N_GRAPHS, EMB))

        @pl.when(i == N_BLKS - 1)
        def _():
            pooled = sums[...] / jnp.maximum(counts[...], 1.0)
            o_ref[...] = (
                jnp.dot(pooled, pw_ref[...], preferred_element_type=F32,
                        precision=HI)
                + pb_ref[...])

    return pl.pallas_call(
        body,
        grid=(N_BLKS,),
        in_specs=[
            pl.BlockSpec((ROW_BLK, EMB), lambda i: (i, 0)),
            pl.BlockSpec((1, 1, ROW_BLK), lambda i: (i, 0, 0)),
            pl.BlockSpec((EMB, OUT), lambda i: (0, 0)),
            pl.BlockSpec((1, OUT), lambda i: (0, 0)),
        ],
        out_specs=pl.BlockSpec((N_GRAPHS, OUT), lambda i: (0, 0)),
        out_shape=jax.ShapeDtypeStruct((N_GRAPHS, OUT), F32),
        scratch_shapes=[
            pltpu.VMEM((N_GRAPHS, EMB), F32),
            pltpu.VMEM((N_GRAPHS, EMB), F32),
        ],
    )(x, batch_r, proj_W, proj_b.reshape(1, -1))


def kernel(x_nodes, edge_index, degrees, identifiers, edge_features, batch,
           atom_emb, id_emb, bond_emb, W1, b1, W2, b2, proj_W, proj_b):
    del degrees

    base = atom_emb[:, 0, :].sum(axis=0).reshape(1, EMB)
    diff = atom_emb[:, 1, :] - atom_emb[:, 0, :]
    d_pad = jnp.zeros((16, EMB), F32).at[:9].set(diff)
    xn_pad = jnp.zeros((N_NODES, 16), F32).at[:, :9].set(
        x_nodes.astype(F32))

    bits = jnp.array([[c & 1, (c >> 1) & 1, (c >> 2) & 1] for c in range(8)],
                     dtype=jnp.int32)
    t_combo = (bond_emb[:, 0, bits[:, 0], :]
               + bond_emb[:, 1, bits[:, 1], :]
               + bond_emb[:, 2, bits[:, 2], :])

    combo = (edge_features[:, 0] + 2 * edge_features[:, 1]
             + 4 * edge_features[:, 2]).astype(jnp.int32)
    src = edge_index[0]
    dst = edge_index[1]

    t0 = (id_emb[:, None, :] + t_combo[0][None, :, :]).reshape(
        ID_VOCAB * 8, EMB)
    cidx0 = identifiers[:, 0] * 8 + combo
    yidx = combo * N_NODES + src

    def pad_tiles(a, nch, cw, pad_val):
        a2 = a.reshape(NW, E_PER_TILE)
        pad = jnp.full((NW, nch * cw - E_PER_TILE), pad_val, jnp.int32)
        return jnp.concatenate([a2, pad], axis=1)

    src_r = pad_tiles(src, NCH0, CW0, 0)
    pcd_r = pad_tiles((cidx0 << PACK_SHIFT) | dst, NCH0, CW0, N_NODES)
    pyd = (yidx << PACK_SHIFT) | dst

    batch_r = batch.reshape(N_BLKS, 1, ROW_BLK)

    sc_l0 = _edge_layer0_call()
    sc_gs = _edge_gather_scatter_call()

    x = _h_kernel(xn_pad, d_pad, base)

    agg = sc_l0(x, t0, src_r, pcd_r)
    x, y = _mlp_kernel(x, agg, W1[0], b1[0], W2[0], b2[0], t_combo[1],
                       relu_out=True, emit_y=True)

    for l in range(1, N_LAYERS):
        agg = sc_gs(y.reshape(8 * N_NODES, EMB), pyd)
        last = l == N_LAYERS - 1
        t_next = t_combo[l + 1] if not last else t_combo[0]
        x, y = _mlp_kernel(x, agg, W1[l], b1[l], W2[l], b2[l], t_next,
                           relu_out=not last, emit_y=not last)

    return _pool_kernel(x, batch_r, proj_W, proj_b)


# speedup vs baseline: 1.5538x; 1.0922x over previous
import functools

import jax
import jax.numpy as jnp
from jax import lax
from jax.experimental import pallas as pl
from jax.experimental.pallas import tpu as pltpu
from jax.experimental.pallas import tpu_sc as plsc

N_NODES = 10000
N_EDGES = 320000
EMB = 128
N_LAYERS = 5
N_GRAPHS = 64
OUT = 128
ID_VOCAB = 50

NC = 2
NS = 16
NW = NC * NS
E_PER_TILE = N_EDGES // NW
CW0 = 80
CC0 = 2 * CW0
NCC0 = N_EDGES // CC0
NCC0_BASE = NCC0 // NW
CW1 = 128
CC1 = 2 * CW1
NCC1 = N_EDGES // CC1
NCC1_BASE = NCC1 // NW
AGG_ROWS = N_NODES + 8
PACK_SHIFT = 14
PACK_MASK = (1 << PACK_SHIFT) - 1
ROWS_PER_TILE = 624
ZTAIL = AGG_ROWS - NS * ROWS_PER_TILE
OTAIL = N_NODES - NS * ROWS_PER_TILE

F32 = jnp.float32
HI = lax.Precision.DEFAULT


def _sc_mesh():
    return plsc.VectorSubcoreMesh(
        core_axis_name="c", subcore_axis_name="s", num_cores=NC, num_subcores=NS)


def _zero_vmem_block(buf, nrows):
    zero = jnp.zeros((16,), F32)

    def row(i, _):
        for k in range(EMB // 16):
            buf[i, pl.ds(k * 16, 16)] = zero
        return 0

    lax.fori_loop(0, nrows, row, 0)


def _zero_agg_stripe(buf, cw, agg, s):
    stripe = s * ROWS_PER_TILE
    n_full = ROWS_PER_TILE // cw
    rem = ROWS_PER_TILE - n_full * cw
    for g in range(n_full):
        pltpu.sync_copy(buf, agg.at[pl.ds(stripe + g * cw, cw)])
    if rem:
        pltpu.sync_copy(buf.at[pl.ds(0, rem)],
                        agg.at[pl.ds(stripe + n_full * cw, rem)])

    @pl.when(s == NS - 1)
    def _():
        pltpu.sync_copy(buf.at[pl.ds(0, ZTAIL)],
                        agg.at[pl.ds(NS * ROWS_PER_TILE, ZTAIL)])


def _copy_out_stripe(agg, out_hbm, c, s):
    stripe = s * ROWS_PER_TILE
    pltpu.sync_copy(agg.at[pl.ds(stripe, ROWS_PER_TILE)],
                    out_hbm.at[c, pl.ds(stripe, ROWS_PER_TILE)])

    @pl.when(s == NS - 1)
    def _():
        pltpu.sync_copy(agg.at[pl.ds(NS * ROWS_PER_TILE, OTAIL)],
                        out_hbm.at[c, pl.ds(NS * ROWS_PER_TILE, OTAIL)])


def _unpack_chunk(packed_i, g, cw, hi_v, lo_v):
    for k in range(cw // 16):
        sl = pl.ds(k * 16, 16)
        v = packed_i[pl.ds(g * cw + k * 16, 16)]
        hi_v[sl] = lax.shift_right_logical(v, PACK_SHIFT)
        lo_v[sl] = lax.bitwise_and(v, PACK_MASK)


def _relu_add_rows(xrows, trows, nrows):
    def row(i, _):
        for kk in range(EMB // 16):
            sl = pl.ds(kk * 16, 16)
            v = xrows[i, sl] + trows[i, sl]
            xrows[i, sl] = jnp.maximum(v, 0.0)
        return 0

    lax.fori_loop(0, nrows, row, 0)


def _edge_layer0_call():
    @functools.partial(
        pl.kernel,
        out_type=jax.ShapeDtypeStruct((NC, N_NODES, EMB), F32),
        mesh=_sc_mesh(),
        scratch_types=[
            pltpu.VMEM((CC0,), jnp.int32),
            pltpu.VMEM((CC0,), jnp.int32),
            pltpu.VMEM((CW0,), jnp.int32),
            pltpu.VMEM((CW0,), jnp.int32),
            pltpu.VMEM((CW0,), jnp.int32),
            pltpu.VMEM((CW0,), jnp.int32),
            pltpu.VMEM((CW0, EMB), F32),
            pltpu.VMEM((CW0, EMB), F32),
            pltpu.VMEM((CW0, EMB), F32),
            pltpu.VMEM((CW0, EMB), F32),
            pltpu.VMEM_SHARED((AGG_ROWS, EMB), F32),
            pltpu.SemaphoreType.DMA,
            pltpu.SemaphoreType.DMA,
        ],
    )
    def k(x_hbm, t_hbm, src_hbm, pcd_hbm, out_hbm,
          sbuf, pbuf, ciA, ciB, diA, diB, xrA, xrB, trA, trB, agg,
          semg, semsc):
        c = lax.axis_index("c")
        s = lax.axis_index("s")
        w = c * NS + s

        _zero_vmem_block(xrA, CW0)
        _zero_agg_stripe(xrA, CW0, agg, s)
        plsc.subcore_barrier()

        n_my = NCC0_BASE + jnp.where(w < NCC0 - NCC0_BASE * NW, 1, 0)

        def body(g, _):
            @pl.when(g > 0)
            def _():
                pltpu.make_async_copy(xrA, agg.at[diA], semsc).wait()
                pltpu.make_async_copy(xrB, agg.at[diB], semsc).wait()

            base = (w + g * NW) * CC0
            pltpu.sync_copy(src_hbm.at[pl.ds(base, CC0)], sbuf)
            pltpu.sync_copy(pcd_hbm.at[pl.ds(base, CC0)], pbuf)
            _unpack_chunk(pbuf, 0, CW0, ciA, diA)
            _unpack_chunk(pbuf, 1, CW0, ciB, diB)
            pltpu.async_copy(x_hbm.at[sbuf.at[pl.ds(0, CW0)]], xrA, semg)
            pltpu.async_copy(x_hbm.at[sbuf.at[pl.ds(CW0, CW0)]], xrB, semg)
            pltpu.async_copy(t_hbm.at[ciA], trA, semg)
            pltpu.async_copy(t_hbm.at[ciB], trB, semg)
            pltpu.make_async_copy(x_hbm.at[sbuf.at[pl.ds(0, CW0)]], xrA,
                                  semg).wait()
            pltpu.make_async_copy(t_hbm.at[ciA], trA, semg).wait()
            _relu_add_rows(xrA, trA, CW0)
            pltpu.make_async_copy(x_hbm.at[sbuf.at[pl.ds(CW0, CW0)]], xrB,
                                  semg).wait()
            pltpu.make_async_copy(t_hbm.at[ciB], trB, semg).wait()
            _relu_add_rows(xrB, trB, CW0)
            pltpu.async_copy(xrA, agg.at[diA], semsc, add=True)
            pltpu.async_copy(xrB, agg.at[diB], semsc, add=True)
            return 0

        lax.fori_loop(0, n_my, body, 0)
        pltpu.make_async_copy(xrA, agg.at[diA], semsc).wait()
        pltpu.make_async_copy(xrB, agg.at[diB], semsc).wait()
        plsc.subcore_barrier()
        _copy_out_stripe(agg, out_hbm, c, s)

    return k


def _edge_gather_scatter_call():
    @functools.partial(
        pl.kernel,
        out_type=jax.ShapeDtypeStruct((NC, N_NODES, EMB), F32),
        mesh=_sc_mesh(),
        scratch_types=[
            pltpu.VMEM((CC1,), jnp.int32),
            pltpu.VMEM((CW1,), jnp.int32),
            pltpu.VMEM((CW1,), jnp.int32),
            pltpu.VMEM((CW1,), jnp.int32),
            pltpu.VMEM((CW1,), jnp.int32),
            pltpu.VMEM((CW1, EMB), F32),
            pltpu.VMEM((CW1, EMB), F32),
            pltpu.VMEM_SHARED((AGG_ROWS, EMB), F32),
            pltpu.SemaphoreType.DMA,
            pltpu.SemaphoreType.DMA,
        ],
    )
    def k(y_hbm, pyd_hbm, out_hbm, pbuf, yiA, yiB, diA, diB, rA, rB, agg,
          semg, semsc):
        c = lax.axis_index("c")
        s = lax.axis_index("s")
        w = c * NS + s

        _zero_vmem_block(rA, CW1)
        _zero_agg_stripe(rA, CW1, agg, s)
        plsc.subcore_barrier()

        n_my = NCC1_BASE + jnp.where(w < NCC1 - NCC1_BASE * NW, 1, 0)

        def body(g, _):
            @pl.when(g > 0)
            def _():
                pltpu.make_async_copy(rA, agg.at[diA], semsc).wait()
                pltpu.make_async_copy(rB, agg.at[diB], semsc).wait()

            base = (w + g * NW) * CC1
            pltpu.sync_copy(pyd_hbm.at[pl.ds(base, CC1)], pbuf)
            _unpack_chunk(pbuf, 0, CW1, yiA, diA)
            _unpack_chunk(pbuf, 1, CW1, yiB, diB)
            pltpu.async_copy(y_hbm.at[yiA], rA, semg)
            pltpu.async_copy(y_hbm.at[yiB], rB, semg)
            pltpu.make_async_copy(y_hbm.at[yiA], rA, semg).wait()
            pltpu.make_async_copy(y_hbm.at[yiB], rB, semg).wait()
            pltpu.async_copy(rA, agg.at[diA], semsc, add=True)
            pltpu.async_copy(rB, agg.at[diB], semsc, add=True)
            return 0

        lax.fori_loop(0, n_my, body, 0)
        pltpu.make_async_copy(rA, agg.at[diA], semsc).wait()
        pltpu.make_async_copy(rB, agg.at[diB], semsc).wait()
        plsc.subcore_barrier()
        _copy_out_stripe(agg, out_hbm, c, s)

    return k


ROW_BLK = 1000
N_BLKS = N_NODES // ROW_BLK


def _h_kernel(xn_pad, d_pad, base):
    def body(xn_ref, d_ref, b_ref, o_ref):
        o_ref[...] = (
            jnp.dot(xn_ref[...], d_ref[...], preferred_element_type=F32,
                    precision=HI)
            + b_ref[...])

    return pl.pallas_call(
        body,
        grid=(N_BLKS,),
        in_specs=[
            pl.BlockSpec((ROW_BLK, 16), lambda i: (i, 0)),
            pl.BlockSpec((16, EMB), lambda i: (0, 0)),
            pl.BlockSpec((1, EMB), lambda i: (0, 0)),
        ],
        out_specs=pl.BlockSpec((ROW_BLK, EMB), lambda i: (i, 0)),
        out_shape=jax.ShapeDtypeStruct((N_NODES, EMB), F32),
    )(xn_pad, d_pad, base)


def _mlp_kernel(x, agg, W1l, b1l, W2l, b2l, t_next, relu_out, emit_y):
    def body(x_ref, a_ref, w1_ref, b1_ref, w2_ref, b2_ref, t_ref,
             xo_ref, yo_ref=None):
        u = x_ref[...] + a_ref[0] + a_ref[1]
        h1 = jnp.maximum(
            jnp.dot(u, w1_ref[...], preferred_element_type=F32, precision=HI)
            + b1_ref[...], 0.0)
        o = (jnp.dot(h1, w2_ref[...], preferred_element_type=F32, precision=HI)
             + b2_ref[...])
        if relu_out:
            o = jnp.maximum(o, 0.0)
        xo_ref[...] = o
        if emit_y:
            for cc in range(8):
                yo_ref[cc] = jnp.maximum(o + t_ref[cc], 0.0)

    out_shapes = [jax.ShapeDtypeStruct((N_NODES, EMB), F32)]
    out_specs = [pl.BlockSpec((ROW_BLK, EMB), lambda i: (i, 0))]
    if emit_y:
        out_shapes.append(jax.ShapeDtypeStruct((8, N_NODES, EMB), F32))
        out_specs.append(pl.BlockSpec((8, ROW_BLK, EMB), lambda i: (0, i, 0)))

    if emit_y:
        wrapped = body
    else:
        def wrapped(x_ref, a_ref, w1_ref, b1_ref, w2_ref, b2_ref, t_ref,
                    xo_ref):
            body(x_ref, a_ref, w1_ref, b1_ref, w2_ref, b2_ref, t_ref, xo_ref)

    res = pl.pallas_call(
        wrapped,
        grid=(N_BLKS,),
        in_specs=[
            pl.BlockSpec((ROW_BLK, EMB), lambda i: (i, 0)),
            pl.BlockSpec((NC, ROW_BLK, EMB), lambda i: (0, i, 0)),
            pl.BlockSpec((EMB, 2 * EMB), lambda i: (0, 0)),
            pl.BlockSpec((1, 2 * EMB), lambda i: (0, 0)),
            pl.BlockSpec((2 * EMB, EMB), lambda i: (0, 0)),
            pl.BlockSpec((1, EMB), lambda i: (0, 0)),
            pl.BlockSpec((8, EMB), lambda i: (0, 0)),
        ],
        out_specs=out_specs,
        out_shape=out_shapes,
    )(x, agg, W1l, b1l.reshape(1, -1), W2l, b2l.reshape(1, -1), t_next)
    if emit_y:
        return res[0], res[1]
    return res[0], None


def _pool_kernel(x, batch_r, proj_W, proj_b):
    def body(x_ref, b_ref, pw_ref, pb_ref, o_ref, sums, counts):
        i = pl.program_id(0)

        @pl.when(i == 0)
        def _():
            sums[...] = jnp.zeros_like(sums)
            counts[...] = jnp.zeros_like(counts)

        gid = lax.broadcasted_iota(jnp.int32, (N_GRAPHS, ROW_BLK), 0)
        oh = (jnp.broadcast_to(b_ref[0], (N_GRAPHS, ROW_BLK)) == gid
              ).astype(F32)
        sums[...] += jnp.dot(oh, x_ref[...], preferred_element_type=F32,
                             precision=HI)
        counts[...] += jnp.broadcast_to(
            jnp.sum(oh, axis=1, keepdims=True), (N_GRAPHS, EMB))

        @pl.when(i == N_BLKS - 1)
        def _():
            pooled = sums[...] / jnp.maximum(counts[...], 1.0)
            o_ref[...] = (
                jnp.dot(pooled, pw_ref[...], preferred_element_type=F32,
                        precision=HI)
                + pb_ref[...])

    return pl.pallas_call(
        body,
        grid=(N_BLKS,),
        in_specs=[
            pl.BlockSpec((ROW_BLK, EMB), lambda i: (i, 0)),
            pl.BlockSpec((1, 1, ROW_BLK), lambda i: (i, 0, 0)),
            pl.BlockSpec((EMB, OUT), lambda i: (0, 0)),
            pl.BlockSpec((1, OUT), lambda i: (0, 0)),
        ],
        out_specs=pl.BlockSpec((N_GRAPHS, OUT), lambda i: (0, 0)),
        out_shape=jax.ShapeDtypeStruct((N_GRAPHS, OUT), F32),
        scratch_shapes=[
            pltpu.VMEM((N_GRAPHS, EMB), F32),
            pltpu.VMEM((N_GRAPHS, EMB), F32),
        ],
    )(x, batch_r, proj_W, proj_b.reshape(1, -1))


def kernel(x_nodes, edge_index, degrees, identifiers, edge_features, batch,
           atom_emb, id_emb, bond_emb, W1, b1, W2, b2, proj_W, proj_b):
    del degrees

    base = atom_emb[:, 0, :].sum(axis=0).reshape(1, EMB)
    diff = atom_emb[:, 1, :] - atom_emb[:, 0, :]
    d_pad = jnp.zeros((16, EMB), F32).at[:9].set(diff)
    xn_pad = jnp.zeros((N_NODES, 16), F32).at[:, :9].set(
        x_nodes.astype(F32))

    bits = jnp.array([[c & 1, (c >> 1) & 1, (c >> 2) & 1] for c in range(8)],
                     dtype=jnp.int32)
    t_combo = (bond_emb[:, 0, bits[:, 0], :]
               + bond_emb[:, 1, bits[:, 1], :]
               + bond_emb[:, 2, bits[:, 2], :])

    combo = (edge_features[:, 0] + 2 * edge_features[:, 1]
             + 4 * edge_features[:, 2]).astype(jnp.int32)
    src = edge_index[0]
    dst = edge_index[1]

    t0 = (id_emb[:, None, :] + t_combo[0][None, :, :]).reshape(
        ID_VOCAB * 8, EMB)
    cidx0 = identifiers[:, 0] * 8 + combo
    yidx = combo * N_NODES + src

    pcd = (cidx0 << PACK_SHIFT) | dst
    pyd = (yidx << PACK_SHIFT) | dst

    batch_r = batch.reshape(N_BLKS, 1, ROW_BLK)

    sc_l0 = _edge_layer0_call()
    sc_gs = _edge_gather_scatter_call()

    x = _h_kernel(xn_pad, d_pad, base)

    agg = sc_l0(x, t0, src, pcd)
    x, y = _mlp_kernel(x, agg, W1[0], b1[0], W2[0], b2[0], t_combo[1],
                       relu_out=True, emit_y=True)

    for l in range(1, N_LAYERS):
        agg = sc_gs(y.reshape(8 * N_NODES, EMB), pyd)
        last = l == N_LAYERS - 1
        t_next = t_combo[l + 1] if not last else t_combo[0]
        x, y = _mlp_kernel(x, agg, W1[l], b1[l], W2[l], b2[l], t_next,
                           relu_out=not last, emit_y=not last)

    return _pool_kernel(x, batch_r, proj_W, proj_b)
